# baseline (speedup 1.0000x reference)
import functools

import jax
import jax.numpy as jnp
from jax.experimental import pallas as pl
from jax.experimental.pallas import tpu as pltpu

_BLK = 4608
_PI = 3.141592653589793


def _acos(x):
    ax = jnp.minimum(jnp.abs(x), 1.0)
    p = jnp.float32(-0.0012624911)
    for c in (0.0066700901, -0.0170881256, 0.0308918810, -0.0501743046,
              0.0889789874, -0.2145988016, 1.5707963050):
        p = p * ax + jnp.float32(c)
    r = jnp.sqrt(1.0 - ax) * p
    return jnp.where(x < 0, jnp.float32(_PI) - r, r)


def _sam_body(o_ref, l_ref, out_ref, acc_ref):
    i = pl.program_id(0)
    o = o_ref[...]
    l = l_ref[...]
    num = jnp.sum(o * l, axis=1)
    oo = jnp.sum(o * o, axis=1)
    ll = jnp.sum(l * l, axis=1)
    ls = jnp.sum(l, axis=1)
    mask = ls != 0.0
    den = jnp.sqrt(oo) * jnp.sqrt(ll)
    ratio = jnp.clip(num / jnp.where(mask, den, 1.0), -1.0, 1.0)
    ang = jnp.where(mask, _acos(ratio), 0.0)
    psum = jnp.sum(ang)
    pcnt = jnp.sum(mask.astype(jnp.float32))

    @pl.when(i == 0)
    def _init():
        acc_ref[0] = 0.0
        acc_ref[1] = 0.0

    acc_ref[0] += psum
    acc_ref[1] += pcnt

    @pl.when(i == pl.num_programs(0) - 1)
    def _fin():
        out_ref[0, 0] = acc_ref[0] / acc_ref[1]


def kernel(outputs, labels):
    b, c, h, w = outputs.shape
    hw = h * w
    o = outputs.reshape(b, c, hw)
    l = labels.reshape(b, c, hw)
    grid = hw // _BLK
    out = pl.pallas_call(
        _sam_body,
        grid=(grid,),
        in_specs=[
            pl.BlockSpec((b, c, _BLK), lambda i: (0, 0, i)),
            pl.BlockSpec((b, c, _BLK), lambda i: (0, 0, i)),
        ],
        out_specs=pl.BlockSpec(memory_space=pltpu.SMEM),
        out_shape=jax.ShapeDtypeStruct((1, 1), jnp.float32),
        scratch_shapes=[pltpu.SMEM((2,), jnp.float32)],
    )(o, l)
    return out[0, 0]


# speedup vs baseline: 4.3229x; 4.3229x over previous
import functools

import jax
import jax.numpy as jnp
from jax.experimental import pallas as pl
from jax.experimental.pallas import tpu as pltpu

_BLK = 4608
_PI = 3.141592653589793


def _acos(x):
    ax = jnp.minimum(jnp.abs(x), 1.0)
    p = jnp.float32(-0.0012624911)
    for c in (0.0066700901, -0.0170881256, 0.0308918810, -0.0501743046,
              0.0889789874, -0.2145988016, 1.5707963050):
        p = p * ax + jnp.float32(c)
    r = jnp.sqrt(1.0 - ax) * p
    return jnp.where(x < 0, jnp.float32(_PI) - r, r)


def _sam_body(o_ref, l_ref, out_ref, acc_ref):
    i = pl.program_id(0)
    o = o_ref[0]
    l = l_ref[0]
    num = jnp.sum(o * l, axis=0)
    oo = jnp.sum(o * o, axis=0)
    ll = jnp.sum(l * l, axis=0)
    ls = jnp.sum(l, axis=0)
    mask = ls != 0.0
    den = jnp.sqrt(oo) * jnp.sqrt(ll)
    ratio = jnp.clip(num / jnp.where(mask, den, 1.0), -1.0, 1.0)
    ang = jnp.where(mask, _acos(ratio), 0.0)
    psum = jnp.sum(ang)
    pcnt = jnp.sum(mask.astype(jnp.float32))

    @pl.when(i == 0)
    def _init():
        acc_ref[0] = 0.0
        acc_ref[1] = 0.0

    acc_ref[0] += psum
    acc_ref[1] += pcnt

    @pl.when(i == pl.num_programs(0) - 1)
    def _fin():
        out_ref[0, 0] = acc_ref[0] / acc_ref[1]


def kernel(outputs, labels):
    b, c, h, w = outputs.shape
    hblk = 16
    nh = h // hblk
    grid = b * nh
    spec = pl.BlockSpec((1, c, hblk, w), lambda i: (i // nh, 0, i % nh, 0))
    out = pl.pallas_call(
        _sam_body,
        grid=(grid,),
        in_specs=[spec, spec],
        out_specs=pl.BlockSpec(memory_space=pltpu.SMEM),
        out_shape=jax.ShapeDtypeStruct((1, 1), jnp.float32),
        scratch_shapes=[pltpu.SMEM((2,), jnp.float32)],
    )(outputs, labels)
    return out[0, 0]
